# baseline (speedup 1.0000x reference)
import functools

import jax
import jax.numpy as jnp
from jax.experimental import pallas as pl
from jax.experimental.pallas import tpu as pltpu
from jax.experimental.pallas import tpu_sc as plsc

T = 2048
D = 2048
H = 16
KVH = 4
HD = 128
E = 8
TOPK = 2
NG = 4
TKG = 2
DFF = 1024
SDFF = 1024
RSF = 1.0
EPS = 1e-6
THETA = 10000.0

BT = 256
BLK = 256
NBLK = T * TOPK // BLK + E
NPAD = NBLK * BLK
NEG = -1e30

f32 = jnp.float32
bf16 = jnp.bfloat16


def _rms(x, w):
    v = jnp.mean(jnp.square(x), axis=-1, keepdims=True)
    return x * jax.lax.rsqrt(v + EPS) * w


def _dot_t(a, b):
    return jax.lax.dot_general(
        a.astype(bf16), b.astype(bf16),
        (((1,), (1,)), ((), ())), preferred_element_type=f32)


def _dot_v(a, b):
    return jax.lax.dot_general(
        a.astype(bf16), b.astype(bf16),
        (((1,), (0,)), ((), ())), preferred_element_type=f32)


def _qkv_body(x_ref, inw_ref, w_ref, o_ref):
    h = _rms(x_ref[...], inw_ref[...])
    o_ref[...] = _dot_t(h, w_ref[...])


def _qkv_call(x, in_ln_w, qkv_w):
    NT = 768
    return pl.pallas_call(
        _qkv_body,
        grid=(4, T // BT),
        in_specs=[
            pl.BlockSpec((BT, D), lambda j, i: (i, 0)),
            pl.BlockSpec((1, D), lambda j, i: (0, 0)),
            pl.BlockSpec((NT, D), lambda j, i: (j, 0)),
        ],
        out_specs=pl.BlockSpec((BT, NT), lambda j, i: (i, j)),
        out_shape=jax.ShapeDtypeStruct((T, (H + 2 * KVH) * HD), f32),
    )(x, in_ln_w.reshape(1, D), qkv_w)


def _rope_body(qkv_ref, qln_ref, kln_ref, cs_ref, q_ref, k_ref, v_ref):
    half = HD // 2
    cos = cs_ref[:, :half]
    sin = cs_ref[:, half:]
    for head in range(H + 2 * KVH):
        s = qkv_ref[:, head * HD:(head + 1) * HD]
        if head < H:
            s = _rms(s, qln_ref[...])
        elif head < H + KVH:
            s = _rms(s, kln_ref[...])
        if head < H + KVH:
            x1 = s[:, :half]
            x2 = s[:, half:]
            s = jnp.concatenate([x1 * cos - x2 * sin, x2 * cos + x1 * sin],
                                axis=1)
        sb = s
        if head < H:
            q_ref[:, head * HD:(head + 1) * HD] = sb
        elif head < H + KVH:
            k_ref[:, (head - H) * HD:(head - H + 1) * HD] = sb
        else:
            v_ref[:, (head - H - KVH) * HD:(head - H - KVH + 1) * HD] = sb


def _rope_call(qkv, q_ln_w, k_ln_w, cs):
    return pl.pallas_call(
        _rope_body,
        grid=(T // BT,),
        in_specs=[
            pl.BlockSpec((BT, (H + 2 * KVH) * HD), lambda i: (i, 0)),
            pl.BlockSpec((1, HD), lambda i: (0, 0)),
            pl.BlockSpec((1, HD), lambda i: (0, 0)),
            pl.BlockSpec((BT, HD), lambda i: (i, 0)),
        ],
        out_specs=[
            pl.BlockSpec((BT, H * HD), lambda i: (i, 0)),
            pl.BlockSpec((BT, KVH * HD), lambda i: (i, 0)),
            pl.BlockSpec((BT, KVH * HD), lambda i: (i, 0)),
        ],
        out_shape=[
            jax.ShapeDtypeStruct((T, H * HD), f32),
            jax.ShapeDtypeStruct((T, KVH * HD), f32),
            jax.ShapeDtypeStruct((T, KVH * HD), f32),
        ],
    )(qkv, q_ln_w.reshape(1, HD), k_ln_w.reshape(1, HD), cs)


def _attn_body(q_ref, k_ref, v_ref, o_ref, s_ref):
    i = pl.program_id(1)
    q = q_ref[...]
    scale = HD ** -0.5
    rows = jax.lax.broadcasted_iota(jnp.int32, (BT, BT), 0)
    cols = jax.lax.broadcasted_iota(jnp.int32, (BT, BT), 1)
    diag_ok = cols <= rows

    def pass1(j, m):
        c = _dot_t(q, k_ref[pl.ds(j * BT, BT), :]) * scale
        c = jnp.where((j == i) & ~diag_ok, NEG, c)
        s_ref[:, pl.ds(j * BT, BT)] = c
        return jnp.maximum(m, jnp.max(c, axis=1, keepdims=True))

    m = jax.lax.fori_loop(0, i + 1, pass1,
                          jnp.full((BT, 1), NEG, f32))

    def pass2(j, carry):
        l, acc = carry
        p = jnp.exp(s_ref[:, pl.ds(j * BT, BT)] - m)
        pv = _dot_v(p, v_ref[pl.ds(j * BT, BT), :])
        return l + jnp.sum(p, axis=1, keepdims=True), acc + pv

    l, acc = jax.lax.fori_loop(
        0, i + 1, pass2,
        (jnp.zeros((BT, 1), f32), jnp.zeros((BT, HD), f32)))
    o_ref[...] = acc / l


def _attn_call(q, k, v):
    rep = H // KVH
    return pl.pallas_call(
        _attn_body,
        grid=(H, T // BT),
        in_specs=[
            pl.BlockSpec((BT, HD), lambda h, i: (i, h)),
            pl.BlockSpec((T, HD), lambda h, i: (0, h // rep)),
            pl.BlockSpec((T, HD), lambda h, i: (0, h // rep)),
        ],
        out_specs=pl.BlockSpec((BT, HD), lambda h, i: (i, h)),
        out_shape=jax.ShapeDtypeStruct((T, H * HD), f32),
        scratch_shapes=[pltpu.VMEM((BT, T), f32)],
    )(q, k, v)


def _oproj_body(a_ref, w_ref, r_ref, x_ref):
    x_ref[...] = _dot_t(a_ref[...], w_ref[...]) + r_ref[...]


def _oproj_call(attn, o_w, resid):
    NT = 1024
    return pl.pallas_call(
        _oproj_body,
        grid=(D // NT, T // BT),
        in_specs=[
            pl.BlockSpec((BT, H * HD), lambda j, i: (i, 0)),
            pl.BlockSpec((NT, H * HD), lambda j, i: (j, 0)),
            pl.BlockSpec((BT, NT), lambda j, i: (i, j)),
        ],
        out_specs=pl.BlockSpec((BT, NT), lambda j, i: (i, j)),
        out_shape=jax.ShapeDtypeStruct((T, D), f32),
    )(attn, o_w, resid)


def _route_body(x_ref, pw_ref, gw_ref, eb_ref, h2_ref, rt_ref):
    h2 = _rms(x_ref[...], pw_ref[...])
    h2_ref[...] = h2.astype(bf16)
    h2r = h2.astype(bf16).astype(f32)
    sig = []
    sfc = []
    for e in range(E):
        gwr = gw_ref[e:e + 1, :].astype(bf16).astype(f32)
        gl = jnp.sum(h2r * gwr, axis=1, keepdims=True)
        s = jax.nn.sigmoid(gl)
        sig.append(s)
        sfc.append(s + eb_ref[0:1, e:e + 1])
    gs = [sfc[2 * g] + sfc[2 * g + 1] for g in range(NG)]
    gok = []
    for g in range(NG):
        cnt = jnp.zeros_like(gs[0], dtype=jnp.int32)
        for g2 in range(NG):
            if g2 == g:
                continue
            beats = (gs[g2] > gs[g]) if g2 > g else (gs[g2] >= gs[g])
            cnt = cnt + beats.astype(jnp.int32)
        gok.append(cnt < TKG)
    mf = [jnp.where(gok[e // 2], sfc[e], NEG) for e in range(E)]
    rank = []
    for e in range(E):
        cnt = jnp.zeros_like(mf[0], dtype=jnp.int32)
        for e2 in range(E):
            if e2 == e:
                continue
            beats = (mf[e2] > mf[e]) if e2 > e else (mf[e2] >= mf[e])
            cnt = cnt + beats.astype(jnp.int32)
        rank.append(cnt)
    zero = jnp.zeros_like(sig[0])
    id0 = zero
    id1 = zero
    w0 = zero
    w1 = zero
    for e in range(E):
        is0 = (rank[e] == 0).astype(f32)
        is1 = (rank[e] == 1).astype(f32)
        id0 = id0 + is0 * e
        id1 = id1 + is1 * e
        w0 = w0 + is0 * sig[e]
        w1 = w1 + is1 * sig[e]
    tot = w0 + w1 + 1e-20
    rt_ref[...] = jnp.concatenate(
        [id0, id1, (w0 / tot) * RSF, (w1 / tot) * RSF], axis=1)


def _route_call(x, post_ln_w, gate_w, e_bias):
    return pl.pallas_call(
        _route_body,
        grid=(T // BT,),
        in_specs=[
            pl.BlockSpec((BT, D), lambda i: (i, 0)),
            pl.BlockSpec((1, D), lambda i: (0, 0)),
            pl.BlockSpec((E, D), lambda i: (0, 0)),
            pl.BlockSpec((1, E), lambda i: (0, 0)),
        ],
        out_specs=[
            pl.BlockSpec((BT, D), lambda i: (i, 0)),
            pl.BlockSpec((BT, 4), lambda i: (i, 0)),
        ],
        out_shape=[
            jax.ShapeDtypeStruct((T, D), bf16),
            jax.ShapeDtypeStruct((T, 4), f32),
        ],
    )(x, post_ln_w.reshape(1, D), gate_w, e_bias.reshape(1, E))


def _shact_body(h2_ref, wg_ref, wu_ref, a_ref):
    h2 = h2_ref[...]
    g = _dot_t(h2, wg_ref[...])
    u = _dot_t(h2, wu_ref[...])
    a_ref[...] = jax.nn.silu(g) * u


def _shact_call(h2, sw_gate_up):
    NT = 512
    return pl.pallas_call(
        _shact_body,
        grid=(SDFF // NT, T // BT),
        in_specs=[
            pl.BlockSpec((BT, D), lambda j, i: (i, 0)),
            pl.BlockSpec((NT, D), lambda j, i: (j, 0)),
            pl.BlockSpec((NT, D), lambda j, i: (j + SDFF // NT, 0)),
        ],
        out_specs=pl.BlockSpec((BT, NT), lambda j, i: (i, j)),
        out_shape=jax.ShapeDtypeStruct((T, SDFF), f32),
    )(h2, sw_gate_up, sw_gate_up)


def _eact_body(be_ref, xg_ref, wg_ref, wu_ref, a_ref):
    x = xg_ref[...]
    g = _dot_t(x, wg_ref[0, 0])
    u = _dot_t(x, wu_ref[0, 0])
    a_ref[...] = jax.nn.silu(g) * u


def _eact_call(block_expert, xg, w_gate_up):
    FT = 512
    NF = DFF // FT
    wgu = w_gate_up.reshape(E, 2 * DFF // FT, FT, D)
    grid_spec = pltpu.PrefetchScalarGridSpec(
        num_scalar_prefetch=1,
        grid=(NF, NBLK),
        in_specs=[
            pl.BlockSpec((BLK, D), lambda f, b, be: (b, 0)),
            pl.BlockSpec((1, 1, FT, D), lambda f, b, be: (be[b], f, 0, 0)),
            pl.BlockSpec((1, 1, FT, D), lambda f, b, be: (be[b], f + NF, 0, 0)),
        ],
        out_specs=pl.BlockSpec((BLK, FT), lambda f, b, be: (b, f)),
    )
    return pl.pallas_call(
        _eact_body,
        grid_spec=grid_spec,
        out_shape=jax.ShapeDtypeStruct((NPAD, DFF), f32),
    )(block_expert, xg, wgu, wgu)


def _edown_body(be_ref, a_ref, wd_ref, y_ref):
    y_ref[...] = _dot_t(a_ref[...], wd_ref[0]).astype(bf16)


def _edown_call(block_expert, act, w_down):
    grid_spec = pltpu.PrefetchScalarGridSpec(
        num_scalar_prefetch=1,
        grid=(NBLK,),
        in_specs=[
            pl.BlockSpec((BLK, DFF), lambda b, be: (b, 0)),
            pl.BlockSpec((1, D, DFF), lambda b, be: (be[b], 0, 0)),
        ],
        out_specs=pl.BlockSpec((BLK, D), lambda b, be: (b, 0)),
    )
    return pl.pallas_call(
        _edown_body,
        grid_spec=grid_spec,
        out_shape=jax.ShapeDtypeStruct((NPAD, D), bf16),
    )(block_expert, act, w_down)


def _sc_gather_impl(data, idx):
    M = idx.shape[1]
    W = 128
    CW = data.shape[1]
    mesh = plsc.VectorSubcoreMesh(core_axis_name="core",
                                  subcore_axis_name="subcore")

    @functools.partial(
        pl.kernel,
        out_type=jax.ShapeDtypeStruct((M, CW), data.dtype),
        mesh=mesh)
    def gk(x_hbm, i_hbm, o_hbm):
        def body(i_vmem, o_vmem):
            pltpu.sync_copy(x_hbm.at[i_vmem.at[0]], o_vmem)

        pltpu.emit_pipeline(
            body,
            grid=(M // W,),
            in_specs=[pl.BlockSpec((1, W), lambda i: (0, i))],
            out_specs=[pl.BlockSpec((W, CW), lambda i: (i, 0))],
            core_axis_name=("core", "subcore"),
            dimension_semantics=(pltpu.PARALLEL,),
        )(i_hbm, o_hbm)

    return gk(data, idx)


def _sc_gather(data, idx, split=8):
    n, c = data.shape
    m = idx.shape[1]
    was_bf16 = data.dtype == bf16
    if was_bf16:
        data = jax.lax.bitcast_convert_type(data.reshape(n, c // 2, 2), f32)
        c = c // 2
        split = split // 2
    cw = c // split
    d2 = data.reshape(n * split, cw)
    idx2 = (idx[0][:, None] * split
            + jnp.arange(split, dtype=jnp.int32)[None, :]).reshape(1, -1)
    out2 = _sc_gather_impl(d2, idx2)
    out = out2.reshape(m, c)
    if was_bf16:
        out = jax.lax.bitcast_convert_type(out, bf16).reshape(m, 2 * c)
    return out


def _comb_body(x_ref, a_ref, wd_ref, rt_ref, y0_ref, y1_ref, o_ref):
    sy = _dot_t(a_ref[...], wd_ref[...])
    w0 = rt_ref[:, 2:3]
    w1 = rt_ref[:, 3:4]
    o_ref[...] = (x_ref[...] + sy + w0 * y0_ref[...].astype(f32)
                  + w1 * y1_ref[...].astype(f32))


def _comb_call(x, a_sh, sw_down, route, yg):
    return pl.pallas_call(
        _comb_body,
        grid=(T // BT,),
        in_specs=[
            pl.BlockSpec((BT, D), lambda i: (i, 0)),
            pl.BlockSpec((BT, SDFF), lambda i: (i, 0)),
            pl.BlockSpec((D, SDFF), lambda i: (0, 0)),
            pl.BlockSpec((BT, 4), lambda i: (i, 0)),
            pl.BlockSpec((BT, D), lambda i: (i, 0)),
            pl.BlockSpec((BT, D), lambda i: (i + T // BT, 0)),
        ],
        out_specs=pl.BlockSpec((BT, D), lambda i: (i, 0)),
        out_shape=jax.ShapeDtypeStruct((T, D), f32),
    )(x, a_sh, sw_down, route, yg, yg)


def _dispatch(route):
    ids = route[:, :TOPK].astype(jnp.int32)
    flat_e = ids.reshape(-1)
    n = T * TOPK
    order = jnp.argsort(flat_e, stable=True)
    sorted_e = flat_e[order]
    counts = jnp.sum((flat_e[None, :] == jnp.arange(E)[:, None]), axis=1)
    padded = ((counts + BLK - 1) // BLK) * BLK
    pad_end = jnp.cumsum(padded)
    pad_start = pad_end - padded
    start = jnp.cumsum(counts) - counts
    rank = jnp.arange(n, dtype=jnp.int32) - start[sorted_e].astype(jnp.int32)
    dest = (pad_start[sorted_e].astype(jnp.int32) + rank)
    sorted_t = (order // TOPK).astype(jnp.int32)
    row_token = jnp.zeros((NPAD,), jnp.int32).at[dest].set(sorted_t)
    inv = jnp.zeros((n,), jnp.int32).at[order].set(dest)
    g01 = inv.reshape(T, TOPK)
    gidx = jnp.concatenate([g01[:, 0], g01[:, 1]]).reshape(1, 2 * T)
    block_expert = jnp.clip(
        jnp.searchsorted(pad_end, jnp.arange(NBLK) * BLK, side="right"),
        0, E - 1).astype(jnp.int32)
    return row_token.reshape(1, NPAD), gidx, block_expert


def kernel(positions, hidden_states, in_ln_w, qkv_w, q_ln_w, k_ln_w, o_w,
           post_ln_w, gate_w, e_bias, w_gate_up, w_down, sw_gate_up, sw_down):
    half = HD // 2
    inv_f = 1.0 / (THETA ** (jnp.arange(half, dtype=f32) / half))
    f = positions.astype(f32)[:, None] * inv_f[None, :]
    cs = jnp.concatenate([jnp.cos(f), jnp.sin(f)], axis=1)

    qkv = _qkv_call(hidden_states, in_ln_w, qkv_w)
    q, k, v = _rope_call(qkv, q_ln_w, k_ln_w, cs)
    attn = _attn_call(q, k, v)
    x = _oproj_call(attn, o_w, hidden_states)
    h2, route = _route_call(x, post_ln_w, gate_w, e_bias)

    row_token, gidx, block_expert = _dispatch(route)
    a_sh = _shact_call(h2, sw_gate_up)
    xg = _sc_gather(h2, row_token)
    act = _eact_call(block_expert, xg, w_gate_up)
    yf = _edown_call(block_expert, act, w_down)
    yg = _sc_gather(yf, gidx)
    return _comb_call(x, a_sh, sw_down, route, yg)


# speedup vs baseline: 1.5541x; 1.5541x over previous
import functools

import jax
import jax.numpy as jnp
from jax.experimental import pallas as pl
from jax.experimental.pallas import tpu as pltpu
from jax.experimental.pallas import tpu_sc as plsc

T = 2048
D = 2048
H = 16
KVH = 4
HD = 128
E = 8
TOPK = 2
NG = 4
TKG = 2
DFF = 1024
SDFF = 1024
RSF = 1.0
EPS = 1e-6
THETA = 10000.0

BT = 256
BLK = 256
NBLK = T * TOPK // BLK + E
NPAD = NBLK * BLK
NEG = -1e30

f32 = jnp.float32
bf16 = jnp.bfloat16


def _rms(x, w):
    v = jnp.mean(jnp.square(x), axis=-1, keepdims=True)
    return x * jax.lax.rsqrt(v + EPS) * w


def _dot_t(a, b):
    return jax.lax.dot_general(
        a.astype(bf16), b.astype(bf16),
        (((1,), (1,)), ((), ())), preferred_element_type=f32)


def _dot_v(a, b):
    return jax.lax.dot_general(
        a.astype(bf16), b.astype(bf16),
        (((1,), (0,)), ((), ())), preferred_element_type=f32)


def _qkv_body(x_ref, inw_ref, w_ref, o_ref):
    h = _rms(x_ref[...], inw_ref[...])
    o_ref[...] = _dot_t(h, w_ref[...])


def _qkv_call(x, in_ln_w, qkv_w):
    NT = 768
    return pl.pallas_call(
        _qkv_body,
        grid=(4, T // BT),
        in_specs=[
            pl.BlockSpec((BT, D), lambda j, i: (i, 0)),
            pl.BlockSpec((1, D), lambda j, i: (0, 0)),
            pl.BlockSpec((NT, D), lambda j, i: (j, 0)),
        ],
        out_specs=pl.BlockSpec((BT, NT), lambda j, i: (i, j)),
        out_shape=jax.ShapeDtypeStruct((T, (H + 2 * KVH) * HD), f32),
    )(x, in_ln_w.reshape(1, D), qkv_w)


def _rope_body(qkv_ref, qln_ref, kln_ref, cs_ref, q_ref, k_ref, v_ref):
    half = HD // 2
    cos = cs_ref[:, :half]
    sin = cs_ref[:, half:]
    for head in range(H + 2 * KVH):
        s = qkv_ref[:, head * HD:(head + 1) * HD]
        if head < H:
            s = _rms(s, qln_ref[...])
        elif head < H + KVH:
            s = _rms(s, kln_ref[...])
        if head < H + KVH:
            x1 = s[:, :half]
            x2 = s[:, half:]
            s = jnp.concatenate([x1 * cos - x2 * sin, x2 * cos + x1 * sin],
                                axis=1)
        sb = s
        if head < H:
            q_ref[:, head * HD:(head + 1) * HD] = sb
        elif head < H + KVH:
            k_ref[:, (head - H) * HD:(head - H + 1) * HD] = sb
        else:
            v_ref[:, (head - H - KVH) * HD:(head - H - KVH + 1) * HD] = sb


def _rope_call(qkv, q_ln_w, k_ln_w, cs):
    return pl.pallas_call(
        _rope_body,
        grid=(T // BT,),
        in_specs=[
            pl.BlockSpec((BT, (H + 2 * KVH) * HD), lambda i: (i, 0)),
            pl.BlockSpec((1, HD), lambda i: (0, 0)),
            pl.BlockSpec((1, HD), lambda i: (0, 0)),
            pl.BlockSpec((BT, HD), lambda i: (i, 0)),
        ],
        out_specs=[
            pl.BlockSpec((BT, H * HD), lambda i: (i, 0)),
            pl.BlockSpec((BT, KVH * HD), lambda i: (i, 0)),
            pl.BlockSpec((BT, KVH * HD), lambda i: (i, 0)),
        ],
        out_shape=[
            jax.ShapeDtypeStruct((T, H * HD), f32),
            jax.ShapeDtypeStruct((T, KVH * HD), f32),
            jax.ShapeDtypeStruct((T, KVH * HD), f32),
        ],
    )(qkv, q_ln_w.reshape(1, HD), k_ln_w.reshape(1, HD), cs)


def _attn_body(q_ref, k_ref, v_ref, o_ref, s_ref):
    i = pl.program_id(1)
    q = q_ref[...]
    scale = HD ** -0.5
    rows = jax.lax.broadcasted_iota(jnp.int32, (BT, BT), 0)
    cols = jax.lax.broadcasted_iota(jnp.int32, (BT, BT), 1)
    diag_ok = cols <= rows

    def pass1(j, m):
        c = _dot_t(q, k_ref[pl.ds(j * BT, BT), :]) * scale
        c = jnp.where((j == i) & ~diag_ok, NEG, c)
        s_ref[:, pl.ds(j * BT, BT)] = c
        return jnp.maximum(m, jnp.max(c, axis=1, keepdims=True))

    m = jax.lax.fori_loop(0, i + 1, pass1,
                          jnp.full((BT, 1), NEG, f32))

    def pass2(j, carry):
        l, acc = carry
        p = jnp.exp(s_ref[:, pl.ds(j * BT, BT)] - m)
        pv = _dot_v(p, v_ref[pl.ds(j * BT, BT), :])
        return l + jnp.sum(p, axis=1, keepdims=True), acc + pv

    l, acc = jax.lax.fori_loop(
        0, i + 1, pass2,
        (jnp.zeros((BT, 1), f32), jnp.zeros((BT, HD), f32)))
    o_ref[...] = acc / l


def _attn_call(q, k, v):
    rep = H // KVH
    return pl.pallas_call(
        _attn_body,
        grid=(H, T // BT),
        in_specs=[
            pl.BlockSpec((BT, HD), lambda h, i: (i, h)),
            pl.BlockSpec((T, HD), lambda h, i: (0, h // rep)),
            pl.BlockSpec((T, HD), lambda h, i: (0, h // rep)),
        ],
        out_specs=pl.BlockSpec((BT, HD), lambda h, i: (i, h)),
        out_shape=jax.ShapeDtypeStruct((T, H * HD), f32),
        scratch_shapes=[pltpu.VMEM((BT, T), f32)],
    )(q, k, v)


def _oproj_body(a_ref, w_ref, r_ref, x_ref):
    x_ref[...] = _dot_t(a_ref[...], w_ref[...]) + r_ref[...]


def _oproj_call(attn, o_w, resid):
    NT = 1024
    return pl.pallas_call(
        _oproj_body,
        grid=(D // NT, T // BT),
        in_specs=[
            pl.BlockSpec((BT, H * HD), lambda j, i: (i, 0)),
            pl.BlockSpec((NT, H * HD), lambda j, i: (j, 0)),
            pl.BlockSpec((BT, NT), lambda j, i: (i, j)),
        ],
        out_specs=pl.BlockSpec((BT, NT), lambda j, i: (i, j)),
        out_shape=jax.ShapeDtypeStruct((T, D), f32),
    )(attn, o_w, resid)


def _route_body(x_ref, pw_ref, gw_ref, eb_ref, h2_ref, rt_ref):
    h2 = _rms(x_ref[...], pw_ref[...])
    h2_ref[...] = h2
    h2r = h2.astype(bf16).astype(f32)
    sig = []
    sfc = []
    for e in range(E):
        gwr = gw_ref[e:e + 1, :].astype(bf16).astype(f32)
        gl = jnp.sum(h2r * gwr, axis=1, keepdims=True)
        s = jax.nn.sigmoid(gl)
        sig.append(s)
        sfc.append(s + eb_ref[0:1, e:e + 1])
    gs = [sfc[2 * g] + sfc[2 * g + 1] for g in range(NG)]
    gok = []
    for g in range(NG):
        cnt = jnp.zeros_like(gs[0], dtype=jnp.int32)
        for g2 in range(NG):
            if g2 == g:
                continue
            beats = (gs[g2] > gs[g]) if g2 > g else (gs[g2] >= gs[g])
            cnt = cnt + beats.astype(jnp.int32)
        gok.append(cnt < TKG)
    mf = [jnp.where(gok[e // 2], sfc[e], NEG) for e in range(E)]
    rank = []
    for e in range(E):
        cnt = jnp.zeros_like(mf[0], dtype=jnp.int32)
        for e2 in range(E):
            if e2 == e:
                continue
            beats = (mf[e2] > mf[e]) if e2 > e else (mf[e2] >= mf[e])
            cnt = cnt + beats.astype(jnp.int32)
        rank.append(cnt)
    zero = jnp.zeros_like(sig[0])
    id0 = zero
    id1 = zero
    w0 = zero
    w1 = zero
    for e in range(E):
        is0 = (rank[e] == 0).astype(f32)
        is1 = (rank[e] == 1).astype(f32)
        id0 = id0 + is0 * e
        id1 = id1 + is1 * e
        w0 = w0 + is0 * sig[e]
        w1 = w1 + is1 * sig[e]
    tot = w0 + w1 + 1e-20
    rt_ref[...] = jnp.concatenate(
        [id0, id1, (w0 / tot) * RSF, (w1 / tot) * RSF], axis=1)


def _route_call(x, post_ln_w, gate_w, e_bias):
    return pl.pallas_call(
        _route_body,
        grid=(T // BT,),
        in_specs=[
            pl.BlockSpec((BT, D), lambda i: (i, 0)),
            pl.BlockSpec((1, D), lambda i: (0, 0)),
            pl.BlockSpec((E, D), lambda i: (0, 0)),
            pl.BlockSpec((1, E), lambda i: (0, 0)),
        ],
        out_specs=[
            pl.BlockSpec((BT, D), lambda i: (i, 0)),
            pl.BlockSpec((BT, 4), lambda i: (i, 0)),
        ],
        out_shape=[
            jax.ShapeDtypeStruct((T, D), f32),
            jax.ShapeDtypeStruct((T, 4), f32),
        ],
    )(x, post_ln_w.reshape(1, D), gate_w, e_bias.reshape(1, E))


def _shact_body(h2_ref, wg_ref, wu_ref, a_ref):
    h2 = h2_ref[...]
    g = _dot_t(h2, wg_ref[...])
    u = _dot_t(h2, wu_ref[...])
    a_ref[...] = jax.nn.silu(g) * u


def _shact_call(h2, sw_gate_up):
    NT = 512
    return pl.pallas_call(
        _shact_body,
        grid=(SDFF // NT, T // BT),
        in_specs=[
            pl.BlockSpec((BT, D), lambda j, i: (i, 0)),
            pl.BlockSpec((NT, D), lambda j, i: (j, 0)),
            pl.BlockSpec((NT, D), lambda j, i: (j + SDFF // NT, 0)),
        ],
        out_specs=pl.BlockSpec((BT, NT), lambda j, i: (i, j)),
        out_shape=jax.ShapeDtypeStruct((T, SDFF), f32),
    )(h2, sw_gate_up, sw_gate_up)


def _eact_body(be_ref, xg_ref, wg_ref, wu_ref, a_ref):
    x = xg_ref[...]
    g = _dot_t(x, wg_ref[0, 0])
    u = _dot_t(x, wu_ref[0, 0])
    a_ref[...] = jax.nn.silu(g) * u


def _eact_call(block_expert, xg, w_gate_up):
    FT = 512
    NF = DFF // FT
    wgu = w_gate_up.reshape(E, 2 * DFF // FT, FT, D)
    grid_spec = pltpu.PrefetchScalarGridSpec(
        num_scalar_prefetch=1,
        grid=(NF, NBLK),
        in_specs=[
            pl.BlockSpec((BLK, D), lambda f, b, be: (b, 0)),
            pl.BlockSpec((1, 1, FT, D), lambda f, b, be: (be[b], f, 0, 0)),
            pl.BlockSpec((1, 1, FT, D), lambda f, b, be: (be[b], f + NF, 0, 0)),
        ],
        out_specs=pl.BlockSpec((BLK, FT), lambda f, b, be: (b, f)),
    )
    return pl.pallas_call(
        _eact_body,
        grid_spec=grid_spec,
        out_shape=jax.ShapeDtypeStruct((NPAD, DFF), f32),
    )(block_expert, xg, wgu, wgu)


def _edown_body(be_ref, a_ref, wd_ref, y_ref):
    y_ref[...] = _dot_t(a_ref[...], wd_ref[0])


def _edown_call(block_expert, act, w_down):
    grid_spec = pltpu.PrefetchScalarGridSpec(
        num_scalar_prefetch=1,
        grid=(NBLK,),
        in_specs=[
            pl.BlockSpec((BLK, DFF), lambda b, be: (b, 0)),
            pl.BlockSpec((1, D, DFF), lambda b, be: (be[b], 0, 0)),
        ],
        out_specs=pl.BlockSpec((BLK, D), lambda b, be: (b, 0)),
    )
    return pl.pallas_call(
        _edown_body,
        grid_spec=grid_spec,
        out_shape=jax.ShapeDtypeStruct((NPAD, D), f32),
    )(block_expert, act, w_down)


def _sc_gather_impl(data, idx):
    M = idx.shape[1]
    W = 128
    CW = data.shape[1]
    mesh = plsc.VectorSubcoreMesh(core_axis_name="core",
                                  subcore_axis_name="subcore")

    @functools.partial(
        pl.kernel,
        out_type=jax.ShapeDtypeStruct((M, CW), data.dtype),
        mesh=mesh)
    def gk(x_hbm, i_hbm, o_hbm):
        def body(i_vmem, o_vmem):
            pltpu.sync_copy(x_hbm.at[i_vmem.at[0]], o_vmem)

        pltpu.emit_pipeline(
            body,
            grid=(M // W,),
            in_specs=[pl.BlockSpec((1, W), lambda i: (0, i))],
            out_specs=[pl.BlockSpec((W, CW), lambda i: (i, 0))],
            core_axis_name=("core", "subcore"),
            dimension_semantics=(pltpu.PARALLEL,),
        )(i_hbm, o_hbm)

    return gk(data, idx)


def _sc_gather(data, idx, split=8):
    n, c = data.shape
    m = idx.shape[1]
    cw = c // split
    d2 = data.reshape(n * split, cw)
    idx2 = (idx[0][:, None] * split
            + jnp.arange(split, dtype=jnp.int32)[None, :]).reshape(1, -1)
    out2 = _sc_gather_impl(d2, idx2)
    return out2.reshape(m, c)


def _sdown_body(x_ref, a_ref, wd_ref, o_ref):
    o_ref[...] = x_ref[...] + _dot_t(a_ref[...], wd_ref[...])


def _sdown_call(x, a_sh, sw_down):
    return pl.pallas_call(
        _sdown_body,
        grid=(T // BT,),
        in_specs=[
            pl.BlockSpec((BT, D), lambda i: (i, 0)),
            pl.BlockSpec((BT, SDFF), lambda i: (i, 0)),
            pl.BlockSpec((D, SDFF), lambda i: (0, 0)),
        ],
        out_specs=pl.BlockSpec((BT, D), lambda i: (i, 0)),
        out_shape=jax.ShapeDtypeStruct((T, D), f32),
    )(x, a_sh, sw_down)


def _comb_body(xs_ref, rt_ref, y0_ref, y1_ref, o_ref):
    w0 = rt_ref[:, 2:3]
    w1 = rt_ref[:, 3:4]
    o_ref[...] = xs_ref[...] + w0 * y0_ref[...] + w1 * y1_ref[...]


def _comb_call(xs, route, yg):
    return pl.pallas_call(
        _comb_body,
        grid=(T // BT,),
        in_specs=[
            pl.BlockSpec((BT, D), lambda i: (i, 0)),
            pl.BlockSpec((BT, 4), lambda i: (i, 0)),
            pl.BlockSpec((BT, D), lambda i: (i, 0)),
            pl.BlockSpec((BT, D), lambda i: (i + T // BT, 0)),
        ],
        out_specs=pl.BlockSpec((BT, D), lambda i: (i, 0)),
        out_shape=jax.ShapeDtypeStruct((T, D), f32),
    )(xs, route, yg, yg)


def _dispatch(route):
    ids = route[:, :TOPK].astype(jnp.int32)
    flat_e = ids.reshape(-1)
    n = T * TOPK
    order = jnp.argsort(flat_e, stable=True)
    sorted_e = flat_e[order]
    counts = jnp.sum((flat_e[None, :] == jnp.arange(E)[:, None]), axis=1)
    padded = ((counts + BLK - 1) // BLK) * BLK
    pad_end = jnp.cumsum(padded)
    pad_start = pad_end - padded
    start = jnp.cumsum(counts) - counts
    rank = jnp.arange(n, dtype=jnp.int32) - start[sorted_e].astype(jnp.int32)
    dest = (pad_start[sorted_e].astype(jnp.int32) + rank)
    sorted_t = (order // TOPK).astype(jnp.int32)
    row_token = jnp.zeros((NPAD,), jnp.int32).at[dest].set(sorted_t)
    inv = jnp.zeros((n,), jnp.int32).at[order].set(dest)
    g01 = inv.reshape(T, TOPK)
    gidx = jnp.concatenate([g01[:, 0], g01[:, 1]]).reshape(1, 2 * T)
    block_expert = jnp.clip(
        jnp.searchsorted(pad_end, jnp.arange(NBLK) * BLK, side="right"),
        0, E - 1).astype(jnp.int32)
    return row_token.reshape(1, NPAD), gidx, block_expert


def kernel(positions, hidden_states, in_ln_w, qkv_w, q_ln_w, k_ln_w, o_w,
           post_ln_w, gate_w, e_bias, w_gate_up, w_down, sw_gate_up, sw_down):
    half = HD // 2
    inv_f = 1.0 / (THETA ** (jnp.arange(half, dtype=f32) / half))
    f = positions.astype(f32)[:, None] * inv_f[None, :]
    cs = jnp.concatenate([jnp.cos(f), jnp.sin(f)], axis=1)

    qkv = _qkv_call(hidden_states, in_ln_w, qkv_w)
    q, k, v = _rope_call(qkv, q_ln_w, k_ln_w, cs)
    attn = _attn_call(q, k, v)
    x = _oproj_call(attn, o_w, hidden_states)
    h2, route = _route_call(x, post_ln_w, gate_w, e_bias)

    row_token, gidx, block_expert = _dispatch(route)
    a_sh = _shact_call(h2, sw_gate_up)
    xg = _sc_gather(h2, row_token)
    xs = _sdown_call(x, a_sh, sw_down)
    act = _eact_call(block_expert, xg, w_gate_up)
    yf = _edown_call(block_expert, act, w_down)
    yg = _sc_gather(yf, gidx)
    return _comb_call(xs, route, yg)


# speedup vs baseline: 1.5700x; 1.0102x over previous
import functools

import jax
import jax.numpy as jnp
from jax.experimental import pallas as pl
from jax.experimental.pallas import tpu as pltpu
from jax.experimental.pallas import tpu_sc as plsc

T = 2048
D = 2048
H = 16
KVH = 4
HD = 128
E = 8
TOPK = 2
NG = 4
TKG = 2
DFF = 1024
SDFF = 1024
RSF = 1.0
EPS = 1e-6
THETA = 10000.0

BT = 256
BLK = 256
NBLK = T * TOPK // BLK + E
NPAD = NBLK * BLK
NEG = -1e30

f32 = jnp.float32
bf16 = jnp.bfloat16


def _rms(x, w):
    v = jnp.mean(jnp.square(x), axis=-1, keepdims=True)
    return x * jax.lax.rsqrt(v + EPS) * w


def _dot_t(a, b):
    return jax.lax.dot_general(
        a.astype(bf16), b.astype(bf16),
        (((1,), (1,)), ((), ())), preferred_element_type=f32)


def _dot_v(a, b):
    return jax.lax.dot_general(
        a.astype(bf16), b.astype(bf16),
        (((1,), (0,)), ((), ())), preferred_element_type=f32)


def _qkv_body(x_ref, inw_ref, w_ref, o_ref):
    h = _rms(x_ref[...], inw_ref[...])
    o_ref[...] = _dot_t(h, w_ref[...])


def _qkv_call(x, in_ln_w, qkv_w):
    NT = 768
    return pl.pallas_call(
        _qkv_body,
        grid=(4, T // BT),
        in_specs=[
            pl.BlockSpec((BT, D), lambda j, i: (i, 0)),
            pl.BlockSpec((1, D), lambda j, i: (0, 0)),
            pl.BlockSpec((NT, D), lambda j, i: (j, 0)),
        ],
        out_specs=pl.BlockSpec((BT, NT), lambda j, i: (i, j)),
        out_shape=jax.ShapeDtypeStruct((T, (H + 2 * KVH) * HD), f32),
    )(x, in_ln_w.reshape(1, D), qkv_w)


def _rope_body(qkv_ref, qln_ref, kln_ref, cs_ref, q_ref, k_ref, v_ref):
    half = HD // 2
    cos = cs_ref[:, :half]
    sin = cs_ref[:, half:]
    for head in range(H + 2 * KVH):
        s = qkv_ref[:, head * HD:(head + 1) * HD]
        if head < H:
            s = _rms(s, qln_ref[...])
        elif head < H + KVH:
            s = _rms(s, kln_ref[...])
        if head < H + KVH:
            x1 = s[:, :half]
            x2 = s[:, half:]
            s = jnp.concatenate([x1 * cos - x2 * sin, x2 * cos + x1 * sin],
                                axis=1)
        sb = s
        if head < H:
            q_ref[:, head * HD:(head + 1) * HD] = sb
        elif head < H + KVH:
            k_ref[:, (head - H) * HD:(head - H + 1) * HD] = sb
        else:
            v_ref[:, (head - H - KVH) * HD:(head - H - KVH + 1) * HD] = sb


def _rope_call(qkv, q_ln_w, k_ln_w, cs):
    return pl.pallas_call(
        _rope_body,
        grid=(T // BT,),
        in_specs=[
            pl.BlockSpec((BT, (H + 2 * KVH) * HD), lambda i: (i, 0)),
            pl.BlockSpec((1, HD), lambda i: (0, 0)),
            pl.BlockSpec((1, HD), lambda i: (0, 0)),
            pl.BlockSpec((BT, HD), lambda i: (i, 0)),
        ],
        out_specs=[
            pl.BlockSpec((BT, H * HD), lambda i: (i, 0)),
            pl.BlockSpec((BT, KVH * HD), lambda i: (i, 0)),
            pl.BlockSpec((BT, KVH * HD), lambda i: (i, 0)),
        ],
        out_shape=[
            jax.ShapeDtypeStruct((T, H * HD), f32),
            jax.ShapeDtypeStruct((T, KVH * HD), f32),
            jax.ShapeDtypeStruct((T, KVH * HD), f32),
        ],
    )(qkv, q_ln_w.reshape(1, HD), k_ln_w.reshape(1, HD), cs)


def _attn_body(q_ref, k_ref, v_ref, o_ref, s_ref):
    i = pl.program_id(1)
    q = q_ref[...]
    scale = HD ** -0.5
    rows = jax.lax.broadcasted_iota(jnp.int32, (BT, BT), 0)
    cols = jax.lax.broadcasted_iota(jnp.int32, (BT, BT), 1)
    diag_ok = cols <= rows

    def pass1(j, m):
        c = _dot_t(q, k_ref[pl.ds(j * BT, BT), :]) * scale
        c = jnp.where((j == i) & ~diag_ok, NEG, c)
        s_ref[:, pl.ds(j * BT, BT)] = c
        return jnp.maximum(m, jnp.max(c, axis=1, keepdims=True))

    m = jax.lax.fori_loop(0, i + 1, pass1,
                          jnp.full((BT, 1), NEG, f32))

    def pass2(j, carry):
        l, acc = carry
        p = jnp.exp(s_ref[:, pl.ds(j * BT, BT)] - m)
        pv = _dot_v(p, v_ref[pl.ds(j * BT, BT), :])
        return l + jnp.sum(p, axis=1, keepdims=True), acc + pv

    l, acc = jax.lax.fori_loop(
        0, i + 1, pass2,
        (jnp.zeros((BT, 1), f32), jnp.zeros((BT, HD), f32)))
    o_ref[...] = acc / l


def _attn_call(q, k, v):
    rep = H // KVH
    return pl.pallas_call(
        _attn_body,
        grid=(H, T // BT),
        in_specs=[
            pl.BlockSpec((BT, HD), lambda h, i: (i, h)),
            pl.BlockSpec((T, HD), lambda h, i: (0, h // rep)),
            pl.BlockSpec((T, HD), lambda h, i: (0, h // rep)),
        ],
        out_specs=pl.BlockSpec((BT, HD), lambda h, i: (i, h)),
        out_shape=jax.ShapeDtypeStruct((T, H * HD), f32),
        scratch_shapes=[pltpu.VMEM((BT, T), f32)],
    )(q, k, v)


def _oproj_body(a_ref, w_ref, r_ref, x_ref):
    x_ref[...] = _dot_t(a_ref[...], w_ref[...]) + r_ref[...]


def _oproj_call(attn, o_w, resid):
    NT = 1024
    return pl.pallas_call(
        _oproj_body,
        grid=(D // NT, T // BT),
        in_specs=[
            pl.BlockSpec((BT, H * HD), lambda j, i: (i, 0)),
            pl.BlockSpec((NT, H * HD), lambda j, i: (j, 0)),
            pl.BlockSpec((BT, NT), lambda j, i: (i, j)),
        ],
        out_specs=pl.BlockSpec((BT, NT), lambda j, i: (i, j)),
        out_shape=jax.ShapeDtypeStruct((T, D), f32),
    )(attn, o_w, resid)


def _route_body(x_ref, pw_ref, gw_ref, eb_ref, h2_ref, rt_ref):
    h2 = _rms(x_ref[...], pw_ref[...])
    h2_ref[...] = h2
    h2r = h2.astype(bf16).astype(f32)
    sig = []
    sfc = []
    for e in range(E):
        gwr = gw_ref[e:e + 1, :].astype(bf16).astype(f32)
        gl = jnp.sum(h2r * gwr, axis=1, keepdims=True)
        s = jax.nn.sigmoid(gl)
        sig.append(s)
        sfc.append(s + eb_ref[0:1, e:e + 1])
    gs = [sfc[2 * g] + sfc[2 * g + 1] for g in range(NG)]
    gok = []
    for g in range(NG):
        cnt = jnp.zeros_like(gs[0], dtype=jnp.int32)
        for g2 in range(NG):
            if g2 == g:
                continue
            beats = (gs[g2] > gs[g]) if g2 > g else (gs[g2] >= gs[g])
            cnt = cnt + beats.astype(jnp.int32)
        gok.append(cnt < TKG)
    mf = [jnp.where(gok[e // 2], sfc[e], NEG) for e in range(E)]
    rank = []
    for e in range(E):
        cnt = jnp.zeros_like(mf[0], dtype=jnp.int32)
        for e2 in range(E):
            if e2 == e:
                continue
            beats = (mf[e2] > mf[e]) if e2 > e else (mf[e2] >= mf[e])
            cnt = cnt + beats.astype(jnp.int32)
        rank.append(cnt)
    zero = jnp.zeros_like(sig[0])
    id0 = zero
    id1 = zero
    w0 = zero
    w1 = zero
    for e in range(E):
        is0 = (rank[e] == 0).astype(f32)
        is1 = (rank[e] == 1).astype(f32)
        id0 = id0 + is0 * e
        id1 = id1 + is1 * e
        w0 = w0 + is0 * sig[e]
        w1 = w1 + is1 * sig[e]
    tot = w0 + w1 + 1e-20
    rt_ref[...] = jnp.concatenate(
        [id0, id1, (w0 / tot) * RSF, (w1 / tot) * RSF], axis=1)


def _route_call(x, post_ln_w, gate_w, e_bias):
    return pl.pallas_call(
        _route_body,
        grid=(T // BT,),
        in_specs=[
            pl.BlockSpec((BT, D), lambda i: (i, 0)),
            pl.BlockSpec((1, D), lambda i: (0, 0)),
            pl.BlockSpec((E, D), lambda i: (0, 0)),
            pl.BlockSpec((1, E), lambda i: (0, 0)),
        ],
        out_specs=[
            pl.BlockSpec((BT, D), lambda i: (i, 0)),
            pl.BlockSpec((BT, 4), lambda i: (i, 0)),
        ],
        out_shape=[
            jax.ShapeDtypeStruct((T, D), f32),
            jax.ShapeDtypeStruct((T, 4), f32),
        ],
    )(x, post_ln_w.reshape(1, D), gate_w, e_bias.reshape(1, E))


def _shact_body(h2_ref, wg_ref, wu_ref, a_ref):
    h2 = h2_ref[...]
    g = _dot_t(h2, wg_ref[...])
    u = _dot_t(h2, wu_ref[...])
    a_ref[...] = jax.nn.silu(g) * u


def _shact_call(h2, sw_gate_up):
    NT = 512
    return pl.pallas_call(
        _shact_body,
        grid=(SDFF // NT, T // BT),
        in_specs=[
            pl.BlockSpec((BT, D), lambda j, i: (i, 0)),
            pl.BlockSpec((NT, D), lambda j, i: (j, 0)),
            pl.BlockSpec((NT, D), lambda j, i: (j + SDFF // NT, 0)),
        ],
        out_specs=pl.BlockSpec((BT, NT), lambda j, i: (i, j)),
        out_shape=jax.ShapeDtypeStruct((T, SDFF), f32),
    )(h2, sw_gate_up, sw_gate_up)


def _eact_body(be_ref, xg_ref, wg_ref, wu_ref, a_ref):
    x = xg_ref[...]
    g = _dot_t(x, wg_ref[0, 0])
    u = _dot_t(x, wu_ref[0, 0])
    a_ref[...] = jax.nn.silu(g) * u


def _eact_call(block_expert, xg, w_gate_up):
    FT = 512
    NF = DFF // FT
    wgu = w_gate_up.reshape(E, 2 * DFF // FT, FT, D)
    grid_spec = pltpu.PrefetchScalarGridSpec(
        num_scalar_prefetch=1,
        grid=(NF, NBLK),
        in_specs=[
            pl.BlockSpec((BLK, D), lambda f, b, be: (b, 0)),
            pl.BlockSpec((1, 1, FT, D), lambda f, b, be: (be[b], f, 0, 0)),
            pl.BlockSpec((1, 1, FT, D), lambda f, b, be: (be[b], f + NF, 0, 0)),
        ],
        out_specs=pl.BlockSpec((BLK, FT), lambda f, b, be: (b, f)),
    )
    return pl.pallas_call(
        _eact_body,
        grid_spec=grid_spec,
        out_shape=jax.ShapeDtypeStruct((NPAD, DFF), f32),
    )(block_expert, xg, wgu, wgu)


def _edown_body(be_ref, a_ref, wd_ref, y_ref):
    y_ref[...] = _dot_t(a_ref[...], wd_ref[0])


def _edown_call(block_expert, act, w_down):
    grid_spec = pltpu.PrefetchScalarGridSpec(
        num_scalar_prefetch=1,
        grid=(NBLK,),
        in_specs=[
            pl.BlockSpec((BLK, DFF), lambda b, be: (b, 0)),
            pl.BlockSpec((1, D, DFF), lambda b, be: (be[b], 0, 0)),
        ],
        out_specs=pl.BlockSpec((BLK, D), lambda b, be: (b, 0)),
    )
    return pl.pallas_call(
        _edown_body,
        grid_spec=grid_spec,
        out_shape=jax.ShapeDtypeStruct((NPAD, D), f32),
    )(block_expert, act, w_down)


def _sc_gather_impl(data, idx):
    M = idx.shape[1]
    W = 128
    CW = data.shape[1]
    mesh = plsc.VectorSubcoreMesh(core_axis_name="core",
                                  subcore_axis_name="subcore")

    @functools.partial(
        pl.kernel,
        out_type=jax.ShapeDtypeStruct((M, CW), data.dtype),
        mesh=mesh)
    def gk(x_hbm, i_hbm, o_hbm):
        def body(i_vmem, o_vmem):
            pltpu.sync_copy(x_hbm.at[i_vmem.at[0]], o_vmem)

        pltpu.emit_pipeline(
            body,
            grid=(M // W,),
            in_specs=[pl.BlockSpec((1, W), lambda i: (0, i))],
            out_specs=[pl.BlockSpec((W, CW), lambda i: (i, 0))],
            core_axis_name=("core", "subcore"),
            dimension_semantics=(pltpu.PARALLEL,),
        )(i_hbm, o_hbm)

    return gk(data, idx)


def _sc_gather(data, idx, split=8):
    n, c = data.shape
    m = idx.shape[1]
    cw = c // split
    d2 = data.reshape(n * split, cw)
    idx2 = (idx[0][:, None] * split
            + jnp.arange(split, dtype=jnp.int32)[None, :]).reshape(1, -1)
    out2 = _sc_gather_impl(d2, idx2)
    return out2.reshape(m, c)


def _comb_body(x_ref, a_ref, wd_ref, rt_ref, y0_ref, y1_ref, o_ref):
    sy = _dot_t(a_ref[...], wd_ref[...])
    w0 = rt_ref[:, 2:3]
    w1 = rt_ref[:, 3:4]
    o_ref[...] = x_ref[...] + sy + w0 * y0_ref[...] + w1 * y1_ref[...]


def _comb_call(x, a_sh, sw_down, route, yg):
    return pl.pallas_call(
        _comb_body,
        grid=(T // BT,),
        in_specs=[
            pl.BlockSpec((BT, D), lambda i: (i, 0)),
            pl.BlockSpec((BT, SDFF), lambda i: (i, 0)),
            pl.BlockSpec((D, SDFF), lambda i: (0, 0)),
            pl.BlockSpec((BT, 4), lambda i: (i, 0)),
            pl.BlockSpec((BT, D), lambda i: (i, 0)),
            pl.BlockSpec((BT, D), lambda i: (i + T // BT, 0)),
        ],
        out_specs=pl.BlockSpec((BT, D), lambda i: (i, 0)),
        out_shape=jax.ShapeDtypeStruct((T, D), f32),
    )(x, a_sh, sw_down, route, yg, yg)


def _dispatch(route):
    ids = route[:, :TOPK].astype(jnp.int32)
    flat_e = ids.reshape(-1)
    n = T * TOPK
    order = jnp.argsort(flat_e, stable=True)
    sorted_e = flat_e[order]
    counts = jnp.sum((flat_e[None, :] == jnp.arange(E)[:, None]), axis=1)
    padded = ((counts + BLK - 1) // BLK) * BLK
    pad_end = jnp.cumsum(padded)
    pad_start = pad_end - padded
    start = jnp.cumsum(counts) - counts
    rank = jnp.arange(n, dtype=jnp.int32) - start[sorted_e].astype(jnp.int32)
    dest = (pad_start[sorted_e].astype(jnp.int32) + rank)
    sorted_t = (order // TOPK).astype(jnp.int32)
    row_token = jnp.zeros((NPAD,), jnp.int32).at[dest].set(sorted_t)
    inv = jnp.zeros((n,), jnp.int32).at[order].set(dest)
    g01 = inv.reshape(T, TOPK)
    gidx = jnp.concatenate([g01[:, 0], g01[:, 1]]).reshape(1, 2 * T)
    block_expert = jnp.clip(
        jnp.searchsorted(pad_end, jnp.arange(NBLK) * BLK, side="right"),
        0, E - 1).astype(jnp.int32)
    return row_token.reshape(1, NPAD), gidx, block_expert


def kernel(positions, hidden_states, in_ln_w, qkv_w, q_ln_w, k_ln_w, o_w,
           post_ln_w, gate_w, e_bias, w_gate_up, w_down, sw_gate_up, sw_down):
    half = HD // 2
    inv_f = 1.0 / (THETA ** (jnp.arange(half, dtype=f32) / half))
    f = positions.astype(f32)[:, None] * inv_f[None, :]
    cs = jnp.concatenate([jnp.cos(f), jnp.sin(f)], axis=1)

    qkv = _qkv_call(hidden_states, in_ln_w, qkv_w)
    q, k, v = _rope_call(qkv, q_ln_w, k_ln_w, cs)
    attn = _attn_call(q, k, v)
    x = _oproj_call(attn, o_w, hidden_states)
    h2, route = _route_call(x, post_ln_w, gate_w, e_bias)

    row_token, gidx, block_expert = _dispatch(route)
    a_sh = _shact_call(h2, sw_gate_up)
    xg = _sc_gather(h2, row_token)
    act = _eact_call(block_expert, xg, w_gate_up)
    yf = _edown_call(block_expert, act, w_down)
    yg = _sc_gather(yf, gidx)
    return _comb_call(x, a_sh, sw_down, route, yg)
